# baseline (speedup 1.0000x reference)
import functools

import jax
import jax.numpy as jnp
from jax import lax
from jax.experimental import pallas as pl
from jax.experimental.pallas import tpu as pltpu
from jax.experimental.pallas import tpu_sc as plsc

NC = 2
NS = 16
NW = NC * NS
EMBED = 64
B = 1024
NIJ = 1024
NBUF = 4
EPART = EMBED // NBUF


def _sc_body(tbl_hbm, x_hbm, out_hbm, tbl_v, x_v, obs, sems):
    wid = lax.axis_index("s") * NC + lax.axis_index("c")
    pairs = NIJ // NW
    base = wid * pairs
    pltpu.sync_copy(tbl_hbm, tbl_v)
    pltpu.sync_copy(x_hbm.at[pl.ds(base, pairs)], x_v)

    def pair(p, carry):
        for q in range(NBUF):
            ob, sem = obs[q], sems[q]
            @pl.when(p > 0)
            def _(ob=ob, sem=sem, q=q):
                pltpu.make_async_copy(
                    ob, out_hbm.at[base + p - 1, pl.ds(q * EPART, EPART)],
                    sem).wait()

            for lb in range(4):
                ms = [x_v[p, pl.ds(lb * 256 + l * 16, 16)] == 1
                      for l in range(16)]

                def ebody(e4, c, ms=ms, q=q, ob=ob, lb=lb):
                    e0 = e4 * 4
                    for k in range(4):
                        t0v = tbl_v[q * EPART + e0 + k]
                        t1v = tbl_v[EMBED + q * EPART + e0 + k]
                        for l in range(16):
                            ob[e0 + k, pl.ds(lb * 256 + l * 16, 16)] = (
                                jnp.where(ms[l], t0v, t1v))
                    return c

                lax.fori_loop(0, EPART // 4, ebody, 0)
            pltpu.async_copy(
                ob, out_hbm.at[base + p, pl.ds(q * EPART, EPART)], sem)
        return carry

    lax.fori_loop(0, pairs, pair, 0)
    for q in range(NBUF):
        pltpu.make_async_copy(
            obs[q], out_hbm.at[base + pairs - 1, pl.ds(q * EPART, EPART)],
            sems[q]).wait()


def _body(tbl_hbm, x_hbm, out_hbm, tbl_v, x_v, ob0, ob1, ob2, ob3,
          sem0, sem1, sem2, sem3):
    _sc_body(tbl_hbm, x_hbm, out_hbm, tbl_v, x_v,
             (ob0, ob1, ob2, ob3), (sem0, sem1, sem2, sem3))


@functools.partial(jax.jit, static_argnames=())
def kernel(tensors, table):
    b, h, w = tensors.shape
    xt = jnp.transpose(tensors, (1, 2, 0)).reshape(h * w, b)
    tblx = jnp.repeat(table[jnp.array([0, 1])].reshape(2 * EMBED, 1), 16,
                      axis=1)

    mesh = plsc.VectorSubcoreMesh(core_axis_name="c", subcore_axis_name="s")
    out = pl.kernel(
        _body,
        out_type=jax.ShapeDtypeStruct((h * w, EMBED, b), jnp.float32),
        mesh=mesh,
        scratch_types=[
            pltpu.VMEM((2 * EMBED, 16), jnp.float32),
            pltpu.VMEM((NIJ // NW, B), jnp.int32),
        ] + [pltpu.VMEM((EPART, B), jnp.float32)] * NBUF
          + [pltpu.SemaphoreType.DMA] * NBUF,
        compiler_params=pltpu.CompilerParams(use_tc_tiling_on_sc=True,
                                             needs_layout_passes=False),
    )(tblx, xt)
    return jnp.transpose(out.reshape(h, w, EMBED, b), (3, 0, 1, 2))


# speedup vs baseline: 1.0900x; 1.0900x over previous
import functools

import jax
import jax.numpy as jnp
from jax import lax
from jax.experimental import pallas as pl
from jax.experimental.pallas import tpu as pltpu
from jax.experimental.pallas import tpu_sc as plsc

NC = 2
NS = 16
NW = NC * NS
EMBED = 64
B = 1024
NIJ = 1024
NBUF = 4
EPART = EMBED // NBUF


def _sc_body(tbl_hbm, x_hbm, out_hbm, tbl_v, x_v, obs, sems):
    wid = lax.axis_index("s") * NC + lax.axis_index("c")
    pairs = NIJ // NW
    base = wid * pairs
    pltpu.sync_copy(tbl_hbm, tbl_v)
    pltpu.sync_copy(x_hbm.at[pl.ds(base, pairs)], x_v)

    def pair(p, carry):
        for q in range(NBUF):
            ob, sem = obs[q], sems[q]
            @pl.when(p > 0)
            def _(ob=ob, sem=sem, q=q):
                pltpu.make_async_copy(
                    ob, out_hbm.at[base + p - 1, pl.ds(q * EPART, EPART)],
                    sem).wait()

            for lb in range(2):
                ms = [x_v[p, pl.ds(lb * 512 + l * 16, 16)] == 1
                      for l in range(32)]

                def ebody(e, c, ms=ms, q=q, ob=ob, lb=lb):
                    t0v = tbl_v[q * EPART + e]
                    t1v = tbl_v[EMBED + q * EPART + e]
                    for l in range(32):
                        ob[e, pl.ds(lb * 512 + l * 16, 16)] = (
                            jnp.where(ms[l], t0v, t1v))
                    return c

                lax.fori_loop(0, EPART, ebody, 0)
            pltpu.async_copy(
                ob, out_hbm.at[base + p, pl.ds(q * EPART, EPART)], sem)
        return carry

    lax.fori_loop(0, pairs, pair, 0)
    for q in range(NBUF):
        pltpu.make_async_copy(
            obs[q], out_hbm.at[base + pairs - 1, pl.ds(q * EPART, EPART)],
            sems[q]).wait()


def _body(tbl_hbm, x_hbm, out_hbm, tbl_v, x_v, ob0, ob1, ob2, ob3,
          sem0, sem1, sem2, sem3):
    _sc_body(tbl_hbm, x_hbm, out_hbm, tbl_v, x_v,
             (ob0, ob1, ob2, ob3), (sem0, sem1, sem2, sem3))


@functools.partial(jax.jit, static_argnames=())
def kernel(tensors, table):
    b, h, w = tensors.shape
    xt = jnp.transpose(tensors, (1, 2, 0)).reshape(h * w, b)
    tblx = jnp.repeat(table[jnp.array([0, 1])].reshape(2 * EMBED, 1), 16,
                      axis=1)

    mesh = plsc.VectorSubcoreMesh(core_axis_name="c", subcore_axis_name="s")
    out = pl.kernel(
        _body,
        out_type=jax.ShapeDtypeStruct((h * w, EMBED, b), jnp.float32),
        mesh=mesh,
        scratch_types=[
            pltpu.VMEM((2 * EMBED, 16), jnp.float32),
            pltpu.VMEM((NIJ // NW, B), jnp.int32),
        ] + [pltpu.VMEM((EPART, B), jnp.float32)] * NBUF
          + [pltpu.SemaphoreType.DMA] * NBUF,
        compiler_params=pltpu.CompilerParams(use_tc_tiling_on_sc=True,
                                             needs_layout_passes=False),
    )(tblx, xt)
    return jnp.transpose(out.reshape(h, w, EMBED, b), (3, 0, 1, 2))


# speedup vs baseline: 1.2051x; 1.1056x over previous
import functools

import jax
import jax.numpy as jnp
from jax import lax
from jax.experimental import pallas as pl
from jax.experimental.pallas import tpu as pltpu
from jax.experimental.pallas import tpu_sc as plsc

NC = 2
NS = 16
NW = NC * NS
EMBED = 64
B = 1024
NIJ = 1024
NBUF = 4
EPART = EMBED // NBUF


def _sc_body(tbl_hbm, x_hbm, out_hbm, tbl_v, x_v, obs, sems):
    wid = lax.axis_index("s") * NC + lax.axis_index("c")
    pairs = NIJ // NW
    base = wid * pairs
    pltpu.sync_copy(tbl_hbm, tbl_v)
    pltpu.sync_copy(x_hbm.at[pl.ds(base, pairs)], x_v)

    def pair(p, carry):
        for q in range(NBUF):
            ob, sem = obs[q], sems[q]
            @pl.when(p > 0)
            def _(ob=ob, sem=sem, q=q):
                pltpu.make_async_copy(
                    ob, out_hbm.at[base + p - 1, pl.ds(q * EPART, EPART)],
                    sem).wait()

            for lb in range(4):
                xf = [x_v[p, pl.ds(lb * 256 + l * 16, 16)].astype(jnp.float32)
                      for l in range(16)]

                def ebody(e, c, xf=xf, q=q, ob=ob, lb=lb):
                    bv = tbl_v[q * EPART + e]
                    dv = tbl_v[EMBED + q * EPART + e]
                    for l in range(16):
                        ob[e, pl.ds(lb * 256 + l * 16, 16)] = bv + xf[l] * dv
                    return c

                lax.fori_loop(0, EPART, ebody, 0)
            pltpu.async_copy(
                ob, out_hbm.at[base + p, pl.ds(q * EPART, EPART)], sem)
        return carry

    lax.fori_loop(0, pairs, pair, 0)
    for q in range(NBUF):
        pltpu.make_async_copy(
            obs[q], out_hbm.at[base + pairs - 1, pl.ds(q * EPART, EPART)],
            sems[q]).wait()


def _body(tbl_hbm, x_hbm, out_hbm, tbl_v, x_v, ob0, ob1, ob2, ob3,
          sem0, sem1, sem2, sem3):
    _sc_body(tbl_hbm, x_hbm, out_hbm, tbl_v, x_v,
             (ob0, ob1, ob2, ob3), (sem0, sem1, sem2, sem3))


@functools.partial(jax.jit, static_argnames=())
def kernel(tensors, table):
    b, h, w = tensors.shape
    xt = jnp.transpose(tensors, (1, 2, 0)).reshape(h * w, b)
    tblx = jnp.repeat(jnp.concatenate([table[1], table[0] - table[1]])
                      .reshape(2 * EMBED, 1), 16, axis=1)

    mesh = plsc.VectorSubcoreMesh(core_axis_name="c", subcore_axis_name="s")
    out = pl.kernel(
        _body,
        out_type=jax.ShapeDtypeStruct((h * w, EMBED, b), jnp.float32),
        mesh=mesh,
        scratch_types=[
            pltpu.VMEM((2 * EMBED, 16), jnp.float32),
            pltpu.VMEM((NIJ // NW, B), jnp.int32),
        ] + [pltpu.VMEM((EPART, B), jnp.float32)] * NBUF
          + [pltpu.SemaphoreType.DMA] * NBUF,
        compiler_params=pltpu.CompilerParams(use_tc_tiling_on_sc=True,
                                             needs_layout_passes=False),
    )(tblx, xt)
    return jnp.transpose(out.reshape(h, w, EMBED, b), (3, 0, 1, 2))


# speedup vs baseline: 1.2942x; 1.0739x over previous
import functools

import jax
import jax.numpy as jnp
from jax import lax
from jax.experimental import pallas as pl
from jax.experimental.pallas import tpu as pltpu
from jax.experimental.pallas import tpu_sc as plsc

NC = 2
NS = 16
NW = NC * NS
EMBED = 64
B = 1024
NIJ = 1024
NBUF = 4
EPART = EMBED // NBUF


def _sc_body(tbl_hbm, x_hbm, out_hbm, tbl_v, x_v, obs, sems):
    wid = lax.axis_index("s") * NC + lax.axis_index("c")
    pairs = NIJ // NW
    base = wid * pairs
    pltpu.sync_copy(tbl_hbm, tbl_v)
    pltpu.sync_copy(x_hbm.at[pl.ds(base, pairs)], x_v)

    def pair(p, carry):
        for q in range(NBUF):
            ob, sem = obs[q], sems[q]
            @pl.when(p > 0)
            def _(ob=ob, sem=sem, q=q):
                pltpu.make_async_copy(
                    ob, out_hbm.at[base + p - 1, pl.ds(q * EPART, EPART)],
                    sem).wait()

            for lb in range(4):
                ms = [x_v[p, pl.ds(lb * 256 + l * 16, 16)] == 1
                      for l in range(16)]

                def ebody(e, c, ms=ms, q=q, ob=ob, lb=lb):
                    t0v = tbl_v[q * EPART + e]
                    t1v = tbl_v[EMBED + q * EPART + e]
                    for l in range(16):
                        ob[e, pl.ds(lb * 256 + l * 16, 16)] = (
                            jnp.where(ms[l], t0v, t1v))
                    return c

                lax.fori_loop(0, EPART, ebody, 0)
            pltpu.async_copy(
                ob, out_hbm.at[base + p, pl.ds(q * EPART, EPART)], sem)
        return carry

    lax.fori_loop(0, pairs, pair, 0)
    for q in range(NBUF):
        pltpu.make_async_copy(
            obs[q], out_hbm.at[base + pairs - 1, pl.ds(q * EPART, EPART)],
            sems[q]).wait()


def _body(tbl_hbm, x_hbm, out_hbm, tbl_v, x_v, ob0, ob1, ob2, ob3,
          sem0, sem1, sem2, sem3):
    _sc_body(tbl_hbm, x_hbm, out_hbm, tbl_v, x_v,
             (ob0, ob1, ob2, ob3), (sem0, sem1, sem2, sem3))


@functools.partial(jax.jit, static_argnames=())
def kernel(tensors, table):
    b, h, w = tensors.shape
    xt = jnp.transpose(tensors, (1, 2, 0)).reshape(h * w, b)
    tblx = jnp.repeat(table[jnp.array([0, 1])].reshape(2 * EMBED, 1), 16,
                      axis=1)

    mesh = plsc.VectorSubcoreMesh(core_axis_name="c", subcore_axis_name="s")
    out = pl.kernel(
        _body,
        out_type=jax.ShapeDtypeStruct((h * w, EMBED, b), jnp.float32),
        mesh=mesh,
        scratch_types=[
            pltpu.VMEM((2 * EMBED, 16), jnp.float32),
            pltpu.VMEM((NIJ // NW, B), jnp.int32),
        ] + [pltpu.VMEM((EPART, B), jnp.float32)] * NBUF
          + [pltpu.SemaphoreType.DMA] * NBUF,
        compiler_params=pltpu.CompilerParams(use_tc_tiling_on_sc=True,
                                             needs_layout_passes=False),
    )(tblx, xt)
    return jnp.transpose(out.reshape(h, w, EMBED, b), (3, 0, 1, 2))


# speedup vs baseline: 1.6353x; 1.2636x over previous
import functools

import jax
import jax.numpy as jnp
from jax import lax
from jax.experimental import pallas as pl
from jax.experimental.pallas import tpu as pltpu
from jax.experimental.pallas import tpu_sc as plsc

NC = 2
NS = 16
NW = NC * NS
EMBED = 64
B = 1024
NIJ = 1024
NBUF = 4
EPART = EMBED // NBUF


def _sc_body(tbl_hbm, x_hbm, out_hbm, tbl_v, x_v, obs, sems):
    wid = lax.axis_index("s") * NC + lax.axis_index("c")
    pairs = NIJ // NW
    base = wid * pairs
    pltpu.sync_copy(tbl_hbm, tbl_v)
    pltpu.sync_copy(x_hbm.at[pl.ds(base, pairs)], x_v)

    def pair(p, carry):
        for q in range(NBUF):
            ob, sem = obs[q], sems[q]
            @pl.when(p > 0)
            def _(ob=ob, sem=sem, q=q):
                pltpu.make_async_copy(
                    ob, out_hbm.at[base + p - 1, pl.ds(q * EPART, EPART)],
                    sem).wait()

            for lb in range(4):
                ms = [x_v[p, pl.ds(lb * 256 + l * 16, 16)] == 1
                      for l in range(16)]

                @plsc.parallel_loop(0, EPART, unroll=2)
                def _(e, ms=ms, q=q, ob=ob, lb=lb):
                    t0v = tbl_v[q * EPART + e]
                    t1v = tbl_v[EMBED + q * EPART + e]
                    for l in range(16):
                        ob[e, pl.ds(lb * 256 + l * 16, 16)] = (
                            jnp.where(ms[l], t0v, t1v))
            pltpu.async_copy(
                ob, out_hbm.at[base + p, pl.ds(q * EPART, EPART)], sem)
        return carry

    lax.fori_loop(0, pairs, pair, 0)
    for q in range(NBUF):
        pltpu.make_async_copy(
            obs[q], out_hbm.at[base + pairs - 1, pl.ds(q * EPART, EPART)],
            sems[q]).wait()


def _body(tbl_hbm, x_hbm, out_hbm, tbl_v, x_v, ob0, ob1, ob2, ob3,
          sem0, sem1, sem2, sem3):
    _sc_body(tbl_hbm, x_hbm, out_hbm, tbl_v, x_v,
             (ob0, ob1, ob2, ob3), (sem0, sem1, sem2, sem3))


@functools.partial(jax.jit, static_argnames=())
def kernel(tensors, table):
    b, h, w = tensors.shape
    xt = jnp.transpose(tensors, (1, 2, 0)).reshape(h * w, b)
    tblx = jnp.repeat(table[jnp.array([0, 1])].reshape(2 * EMBED, 1), 16,
                      axis=1)

    mesh = plsc.VectorSubcoreMesh(core_axis_name="c", subcore_axis_name="s")
    out = pl.kernel(
        _body,
        out_type=jax.ShapeDtypeStruct((h * w, EMBED, b), jnp.float32),
        mesh=mesh,
        scratch_types=[
            pltpu.VMEM((2 * EMBED, 16), jnp.float32),
            pltpu.VMEM((NIJ // NW, B), jnp.int32),
        ] + [pltpu.VMEM((EPART, B), jnp.float32)] * NBUF
          + [pltpu.SemaphoreType.DMA] * NBUF,
        compiler_params=pltpu.CompilerParams(use_tc_tiling_on_sc=True,
                                             needs_layout_passes=False),
    )(tblx, xt)
    return jnp.transpose(out.reshape(h, w, EMBED, b), (3, 0, 1, 2))
